# baseline (speedup 1.0000x reference)
import functools
import jax
import jax.numpy as jnp
from jax import lax
from jax.experimental import pallas as pl
from jax.experimental.pallas import tpu as pltpu
from jax.experimental.pallas import tpu_sc as plsc

B_, L_, DIM_ = 4, 4096, 1024
NC, NS, LANES = 2, 16, 16
NW = NC * NS
L_PER_W = L_ // NW
R_ = 16


def _sc_body(x_hbm, emb_hbm, out_hbm, emb_v, xb_v):
    wid = lax.axis_index("s") * NC + lax.axis_index("c")
    nchunks = L_PER_W // R_

    def chunk(c, carry):
        base = wid * L_PER_W + c * R_
        pltpu.sync_copy(emb_hbm.at[pl.ds(base, R_)], emb_v)
        for b in range(B_):
            pltpu.sync_copy(x_hbm.at[b, pl.ds(base, R_)], xb_v.at[b])

        def rbody(r, rcarry):
            for j in range(DIM_ // LANES):
                sl = pl.ds(j * LANES, LANES)
                v = emb_v[r, sl]
                for b in range(B_):
                    plsc.addupdate(xb_v.at[b, r, sl], v)
            return rcarry

        lax.fori_loop(0, R_, rbody, 0)
        for b in range(B_):
            pltpu.sync_copy(xb_v.at[b], out_hbm.at[b, pl.ds(base, R_)])
        return carry

    lax.fori_loop(0, nchunks, chunk, 0)


def _sc_add(x, emb):
    mesh = plsc.VectorSubcoreMesh(core_axis_name="c", subcore_axis_name="s")
    return pl.kernel(
        _sc_body,
        out_type=jax.ShapeDtypeStruct((B_, L_, DIM_), jnp.float32),
        mesh=mesh,
        scratch_types=[
            pltpu.VMEM((R_, DIM_), jnp.float32),
            pltpu.VMEM((B_, R_, DIM_), jnp.float32),
        ],
    )(x, emb)


def kernel(x, emb):
    return _sc_add(x, emb)
